# baseline (speedup 1.0000x reference)
import functools

import jax
import jax.numpy as jnp
from jax import lax
from jax.experimental import pallas as pl
from jax.experimental.pallas import tpu as pltpu
from jax.experimental.pallas import tpu_sc as plsc

B = 16384
D = 16
NCOL = 6
_NC = 2
_NS = 16
_NW = _NC * _NS
_BPW = B // _NW
_BF = (False, False, False, False, False, False)


def _sc_gather(tables, indices):
  mesh = plsc.VectorSubcoreMesh(core_axis_name="c", subcore_axis_name="s")

  @functools.partial(
      pl.kernel,
      mesh=mesh,
      compiler_params=pltpu.CompilerParams(use_tc_tiling_on_sc=False),
      out_type=tuple(
          jax.ShapeDtypeStruct((B, D), jnp.bfloat16 if _BF[j] else jnp.float32)
          for j in range(NCOL)),
      scratch_types=(
          [pltpu.VMEM((_BPW,), jnp.int32) for _ in range(NCOL)]
          + [pltpu.VMEM((_BPW, D), jnp.bfloat16 if _BF[j] else jnp.float32)
             for j in range(NCOL)]
          + [pltpu.SemaphoreType.DMA for _ in range(NCOL)]
      ),
  )
  def k(*refs):
    tabs = refs[0:NCOL]
    idxs = refs[NCOL:2 * NCOL]
    outs = refs[2 * NCOL:3 * NCOL]
    idx_v = refs[3 * NCOL:4 * NCOL]
    rows_v = refs[4 * NCOL:5 * NCOL]
    sems = refs[5 * NCOL:6 * NCOL]
    wid = lax.axis_index("s") * _NC + lax.axis_index("c")
    base = wid * _BPW
    for j in range(NCOL):
      pltpu.sync_copy(idxs[j].at[pl.ds(base, _BPW)], idx_v[j])
    cps = [
        pltpu.async_copy(tabs[j].at[idx_v[j]], rows_v[j], sems[j])
        for j in range(NCOL)
    ]
    for j in range(NCOL):
      cps[j].wait()
      pltpu.sync_copy(rows_v[j], outs[j].at[pl.ds(base, _BPW)])

  return k(*tables, *indices)


_V = 1000000
_TCOL = 16384
_TGRID = (_V + _TCOL - 1) // _TCOL
_KB = _TCOL // 8
_VPAD = _TGRID * _TCOL


def _tr_body(a, b, oa, ob):
  col = lax.broadcasted_iota(jnp.int32, (128, 128), 1)
  row = lax.broadcasted_iota(jnp.int32, (128, 128), 0)
  eye = (col == row).astype(jnp.float32)
  for x, o in ((a, oa), (b, ob)):
    xv = x[...]
    xc = jnp.concatenate(
        [xv[:, s * _KB:(s + 1) * _KB] for s in range(8)], axis=0)
    o[...] = lax.dot_general(xc, eye, (((0,), (0,)), ((), ())),
                             preferred_element_type=jnp.float32)


def _tc_transpose(t0, t1):
  ispec = pl.BlockSpec((16, _TCOL), lambda g: (0, g))
  ospec = pl.BlockSpec((_KB, 128), lambda g: (g, 0))
  return pl.pallas_call(
      _tr_body,
      grid=(_TGRID,),
      in_specs=[ispec, ispec],
      out_specs=[ospec, ospec],
      out_shape=[jax.ShapeDtypeStruct((_VPAD // 8, 128), jnp.float32)] * 2,
      compiler_params=pltpu.CompilerParams(fuse_transposed_lhs_in_matmul=True),
  )(t0, t1)


def _pack_idx(i):
  r_local = i % _TCOL
  return (i - r_local) + (r_local % _KB) * 8 + r_local // _KB


_R = 2048


def _mlp_body(e0, e1, e2, e3, e4, e5, w0, b0, w1, b1, w2, b2, out):
  es = (e0, e1, e2, e3, e4, e5)
  w0v = w0[...]
  s = None
  for j in range(NCOL):
    x = es[j][...].astype(jnp.float32)
    p = jnp.dot(x, w0v[D * j:D * (j + 1), :],
                preferred_element_type=jnp.float32)
    s = p if s is None else s + p
  h0 = jnp.maximum(s + b0[...], 0.0)
  h1 = jnp.maximum(
      jnp.dot(h0, w1[...], preferred_element_type=jnp.float32) + b1[...], 0.0)
  out[...] = jnp.dot(h1, w2[...], preferred_element_type=jnp.float32) + b2[...]


def _tc_mlp(embs, W0, b0, W1, b1, W2, b2):
  especs = [pl.BlockSpec((_R, D), lambda g: (g, 0)) for _ in range(NCOL)]

  def wspec(shape):
    return pl.BlockSpec(shape, lambda g: (0, 0))

  return pl.pallas_call(
      _mlp_body,
      grid=(B // _R,),
      in_specs=(
          especs
          + [wspec((D * NCOL, 256)), wspec((1, 256)),
             wspec((256, 128)), wspec((1, 128)),
             wspec((128, 1)), wspec((1, 1))]
      ),
      out_specs=pl.BlockSpec((_R, 1), lambda g: (g, 0)),
      out_shape=jax.ShapeDtypeStruct((B, 1), jnp.float32),
  )(*embs, W0, b0.reshape(1, -1), W1, b1.reshape(1, -1), W2,
    b2.reshape(1, -1))


def kernel(idx_user_id, table_user_id, idx_item_id, table_item_id,
           idx_device, table_device, idx_geo, table_geo,
           idx_hour, table_hour, idx_dayofweek, table_dayofweek,
           W0, b0, W1, b1, W2, b2):
  p0, p1 = _tc_transpose(table_user_id.T, table_item_id.T)
  tables = [p0.reshape(_VPAD, D), p1.reshape(_VPAD, D), table_device,
            table_geo, table_hour, table_dayofweek]
  i0 = _pack_idx(idx_user_id.astype(jnp.int32))
  i1 = _pack_idx(idx_item_id.astype(jnp.int32))
  indices = (i0, i1) + tuple(
      i.astype(jnp.int32)
      for i in (idx_device, idx_geo, idx_hour, idx_dayofweek))
  embs = _sc_gather(tables, indices)
  return _tc_mlp(embs, W0, b0, W1, b1, W2, b2)
